# Initial kernel scaffold; baseline (speedup 1.0000x reference)
import jax
import jax.numpy as jnp
from jax.experimental import pallas as pl


def kernel(bucket_mtx, embeddings_table):
    raise NotImplementedError("write your pallas kernel here")


# speedup vs baseline: 5.8078x; 5.8078x over previous
import functools

import jax
import jax.numpy as jnp
from jax import lax
from jax.experimental import pallas as pl
from jax.experimental.pallas import tpu as pltpu
from jax.experimental.pallas import tpu_sc as plsc

L_SIZE = 2048
EMB = 16
NROWS = L_SIZE * L_SIZE
NW = 32
CHUNK = 2048
PER_W = NROWS // NW
NCH = PER_W // CHUNK
GROUPS = CHUNK // 16


def _sc_lookup(bucket_flat, table_flat):
  mesh = plsc.VectorSubcoreMesh(core_axis_name="c", subcore_axis_name="s")

  @functools.partial(
      pl.kernel,
      mesh=mesh,
      out_type=jax.ShapeDtypeStruct((NROWS * EMB,), jnp.float32),
      compiler_params=pltpu.CompilerParams(needs_layout_passes=False),
      scratch_types=[
          pltpu.VMEM((9 * EMB,), jnp.float32),
          pltpu.VMEM((CHUNK,), jnp.int32),
          pltpu.VMEM((CHUNK,), jnp.int32),
          pltpu.VMEM((CHUNK * EMB,), jnp.float32),
          pltpu.VMEM((CHUNK * EMB,), jnp.float32),
          pltpu.SemaphoreType.DMA,
          pltpu.SemaphoreType.DMA,
      ],
  )
  def k(idx_hbm, tab_hbm, out_hbm, tab_v, idx_v0, idx_v1, out_v0, out_v1,
        sem0, sem1):
    wid = lax.axis_index("s") * 2 + lax.axis_index("c")
    wbase = wid * PER_W
    idx_bufs = (idx_v0, idx_v1)
    out_bufs = (out_v0, out_v1)
    sems = (sem0, sem1)

    pltpu.sync_copy(tab_hbm, tab_v)
    lane = lax.iota(jnp.int32, 16)

    def compute_chunk(slot):
      def grp(g2, carry):
        base = g2 * 16
        v_idx = idx_bufs[slot][pl.ds(base, 16)]
        src = v_idx * EMB
        dst = (base + lane) * EMB
        for col in range(EMB):
          vals = plsc.load_gather(tab_v, [src + col])
          plsc.store_scatter(out_bufs[slot], [dst + col], vals)
        return carry

      lax.fori_loop(0, GROUPS, grp, 0)

    def body(t, carry):
      for slot in range(2):
        g = t * 2 + slot
        rb = (wbase + g * CHUNK) * EMB

        @pl.when(t > 0)
        def _wait():
          prb = rb - 2 * CHUNK * EMB
          pltpu.make_async_copy(
              out_bufs[slot], out_hbm.at[pl.ds(prb, CHUNK * EMB)], sems[slot]
          ).wait()

        pltpu.sync_copy(
            idx_hbm.at[pl.ds(wbase + g * CHUNK, CHUNK)], idx_bufs[slot]
        )
        compute_chunk(slot)
        pltpu.async_copy(
            out_bufs[slot], out_hbm.at[pl.ds(rb, CHUNK * EMB)], sems[slot]
        )
      return carry

    lax.fori_loop(0, NCH // 2, body, 0)

    for slot in range(2):
      rb = (wbase + (NCH - 2 + slot) * CHUNK) * EMB
      pltpu.make_async_copy(
          out_bufs[slot], out_hbm.at[pl.ds(rb, CHUNK * EMB)], sems[slot]
      ).wait()

  return k(bucket_flat, table_flat)


def kernel(bucket_mtx, embeddings_table):
  flat = bucket_mtx.reshape(NROWS)
  out = _sc_lookup(flat, embeddings_table.reshape(9 * EMB))
  return out.reshape(L_SIZE, L_SIZE, EMB)


# speedup vs baseline: 7.2755x; 1.2527x over previous
import functools

import jax
import jax.numpy as jnp
from jax import lax
from jax.experimental import pallas as pl
from jax.experimental.pallas import tpu as pltpu
from jax.experimental.pallas import tpu_sc as plsc

L_SIZE = 2048
EMB = 16
NROWS = L_SIZE * L_SIZE
NW = 32
CHUNK = 2048
PER_W = NROWS // NW
NCH = PER_W // CHUNK
GROUPS = CHUNK // 16


def _sc_lookup(bucket_flat, table_flat):
  mesh = plsc.VectorSubcoreMesh(core_axis_name="c", subcore_axis_name="s")

  @functools.partial(
      pl.kernel,
      mesh=mesh,
      out_type=jax.ShapeDtypeStruct((NROWS * EMB,), jnp.float32),
      compiler_params=pltpu.CompilerParams(needs_layout_passes=False),
      scratch_types=[
          pltpu.VMEM((9 * EMB,), jnp.float32),
          pltpu.VMEM((CHUNK,), jnp.int32),
          pltpu.VMEM((CHUNK,), jnp.int32),
          pltpu.VMEM((CHUNK * EMB,), jnp.float32),
          pltpu.VMEM((CHUNK * EMB,), jnp.float32),
          pltpu.SemaphoreType.DMA,
          pltpu.SemaphoreType.DMA,
      ],
  )
  def k(idx_hbm, tab_hbm, out_hbm, tab_v, idx_v0, idx_v1, out_v0, out_v1,
        sem0, sem1):
    wid = lax.axis_index("s") * 2 + lax.axis_index("c")
    wbase = wid * PER_W
    idx_bufs = (idx_v0, idx_v1)
    out_bufs = (out_v0, out_v1)
    sems = (sem0, sem1)

    pltpu.sync_copy(tab_hbm, tab_v)
    lane = lax.iota(jnp.int32, 16)

    def compute_chunk(slot):
      def grp(g2, carry):
        base = g2 * 16
        v_idx = idx_bufs[slot][pl.ds(base, 16)]
        src = v_idx * EMB
        for r in range(16):
          rsel = jnp.full((16,), r, jnp.int32)
          bsplat = jnp.take_along_axis(src, rsel, axis=0)
          row = plsc.load_gather(tab_v, [bsplat + lane])
          out_bufs[slot][pl.ds((base + r) * EMB, EMB)] = row
        return carry

      lax.fori_loop(0, GROUPS, grp, 0)

    def body(t, carry):
      for slot in range(2):
        g = t * 2 + slot
        rb = (wbase + g * CHUNK) * EMB

        @pl.when(t > 0)
        def _wait():
          prb = rb - 2 * CHUNK * EMB
          pltpu.make_async_copy(
              out_bufs[slot], out_hbm.at[pl.ds(prb, CHUNK * EMB)], sems[slot]
          ).wait()

        pltpu.sync_copy(
            idx_hbm.at[pl.ds(wbase + g * CHUNK, CHUNK)], idx_bufs[slot]
        )
        compute_chunk(slot)
        pltpu.async_copy(
            out_bufs[slot], out_hbm.at[pl.ds(rb, CHUNK * EMB)], sems[slot]
        )
      return carry

    lax.fori_loop(0, NCH // 2, body, 0)

    for slot in range(2):
      rb = (wbase + (NCH - 2 + slot) * CHUNK) * EMB
      pltpu.make_async_copy(
          out_bufs[slot], out_hbm.at[pl.ds(rb, CHUNK * EMB)], sems[slot]
      ).wait()

  return k(bucket_flat, table_flat)


def kernel(bucket_mtx, embeddings_table):
  flat = bucket_mtx.reshape(NROWS)
  out = _sc_lookup(flat, embeddings_table.reshape(9 * EMB))
  return out.reshape(L_SIZE, L_SIZE, EMB)


# speedup vs baseline: 7.3528x; 1.0106x over previous
import functools

import jax
import jax.numpy as jnp
from jax import lax
from jax.experimental import pallas as pl
from jax.experimental.pallas import tpu as pltpu
from jax.experimental.pallas import tpu_sc as plsc

L_SIZE = 2048
EMB = 16
NROWS = L_SIZE * L_SIZE
NW = 32
CHUNK = L_SIZE
MROWS_PER_W = L_SIZE // NW
GROUPS = CHUNK // 16


def _sc_lookup(bucket_mtx, table):
  mesh = plsc.VectorSubcoreMesh(core_axis_name="c", subcore_axis_name="s")

  @functools.partial(
      pl.kernel,
      mesh=mesh,
      out_type=jax.ShapeDtypeStruct((NROWS * EMB,), jnp.float32),
      compiler_params=pltpu.CompilerParams(needs_layout_passes=False),
      scratch_types=[
          pltpu.VMEM((9, EMB), jnp.float32),
          pltpu.VMEM((9 * EMB,), jnp.float32),
          pltpu.VMEM((1, CHUNK), jnp.int32),
          pltpu.VMEM((1, CHUNK), jnp.int32),
          pltpu.VMEM((CHUNK * EMB,), jnp.float32),
          pltpu.VMEM((CHUNK * EMB,), jnp.float32),
          pltpu.SemaphoreType.DMA,
          pltpu.SemaphoreType.DMA,
      ],
  )
  def k(idx_hbm, tab_hbm, out_hbm, tab2d_v, tab_v, idx_v0, idx_v1,
        out_v0, out_v1, sem0, sem1):
    wid = lax.axis_index("s") * 2 + lax.axis_index("c")
    wbase = wid * MROWS_PER_W
    idx_bufs = (idx_v0, idx_v1)
    out_bufs = (out_v0, out_v1)
    sems = (sem0, sem1)

    pltpu.sync_copy(tab_hbm, tab2d_v)
    for e in range(9):
      tab_v[pl.ds(e * EMB, EMB)] = tab2d_v[e, :]
    lane = lax.iota(jnp.int32, 16)

    def compute_chunk(slot):
      def grp(g2, carry):
        base = g2 * 16
        v_idx = idx_bufs[slot][0, pl.ds(base, 16)]
        src = v_idx * EMB
        for r in range(16):
          rsel = jnp.full((16,), r, jnp.int32)
          bsplat = jnp.take_along_axis(src, rsel, axis=0)
          row = plsc.load_gather(tab_v, [bsplat + lane])
          out_bufs[slot][pl.ds((base + r) * EMB, EMB)] = row
        return carry

      lax.fori_loop(0, GROUPS, grp, 0)

    def body(t, carry):
      for slot in range(2):
        mrow = wbase + t * 2 + slot
        rb = mrow * CHUNK * EMB

        @pl.when(t > 0)
        def _wait():
          prb = rb - 2 * CHUNK * EMB
          pltpu.make_async_copy(
              out_bufs[slot], out_hbm.at[pl.ds(prb, CHUNK * EMB)], sems[slot]
          ).wait()

        pltpu.sync_copy(idx_hbm.at[pl.ds(mrow, 1)], idx_bufs[slot])
        compute_chunk(slot)
        pltpu.async_copy(
            out_bufs[slot], out_hbm.at[pl.ds(rb, CHUNK * EMB)], sems[slot]
        )
      return carry

    lax.fori_loop(0, MROWS_PER_W // 2, body, 0)

    for slot in range(2):
      rb = (wbase + MROWS_PER_W - 2 + slot) * CHUNK * EMB
      pltpu.make_async_copy(
          out_bufs[slot], out_hbm.at[pl.ds(rb, CHUNK * EMB)], sems[slot]
      ).wait()

  return k(bucket_mtx, table)


def kernel(bucket_mtx, embeddings_table):
  out = _sc_lookup(bucket_mtx, embeddings_table)
  return out.reshape(L_SIZE, L_SIZE, EMB)


# speedup vs baseline: 77.8764x; 10.5914x over previous
import functools

import jax
import jax.numpy as jnp
from jax import lax
from jax.experimental import pallas as pl
from jax.experimental.pallas import tpu as pltpu
from jax.experimental.pallas import tpu_sc as plsc

L_SIZE = 2048
EMB = 16
NW = 32
CHUNK = L_SIZE
MROWS_PER_W = L_SIZE // NW
GROUPS = CHUNK // 16


def _sc_lookup(bucket_mtx, table):
  mesh = plsc.VectorSubcoreMesh(core_axis_name="c", subcore_axis_name="s")

  @functools.partial(
      pl.kernel,
      mesh=mesh,
      out_type=jax.ShapeDtypeStruct((L_SIZE, EMB, L_SIZE), jnp.float32),
      compiler_params=pltpu.CompilerParams(needs_layout_passes=False),
      scratch_types=[
          pltpu.VMEM((9, EMB), jnp.float32),
          pltpu.VMEM((16 * EMB,), jnp.float32),
          pltpu.VMEM((1, CHUNK), jnp.int32),
          pltpu.VMEM((1, CHUNK), jnp.int32),
          pltpu.VMEM((1, EMB, CHUNK), jnp.float32),
          pltpu.VMEM((1, EMB, CHUNK), jnp.float32),
          pltpu.SemaphoreType.DMA,
          pltpu.SemaphoreType.DMA,
      ],
  )
  def k(idx_hbm, tab_hbm, out_hbm, tab2d_v, tab_flat, idx_v0, idx_v1,
        out_v0, out_v1, sem0, sem1):
    wid = lax.axis_index("s") * 2 + lax.axis_index("c")
    wbase = wid * MROWS_PER_W
    idx_bufs = (idx_v0, idx_v1)
    out_bufs = (out_v0, out_v1)
    sems = (sem0, sem1)

    pltpu.sync_copy(tab_hbm, tab2d_v)
    for e in range(9):
      tab_flat[pl.ds(e * EMB, EMB)] = tab2d_v[e, :]
    lane = lax.iota(jnp.int32, 16)
    tcols = [
        plsc.load_gather(tab_flat, [lane * EMB + k]) for k in range(EMB)
    ]

    def compute_chunk(slot):
      def grp(g2, carry):
        j0 = g2 * 16
        v_idx = idx_bufs[slot][0, pl.ds(j0, 16)]
        for k in range(EMB):
          col = jnp.take_along_axis(tcols[k], v_idx, axis=0)
          out_bufs[slot][0, k, pl.ds(j0, 16)] = col
        return carry

      lax.fori_loop(0, GROUPS, grp, 0)

    def body(t, carry):
      for slot in range(2):
        mrow = wbase + t * 2 + slot

        @pl.when(t > 0)
        def _wait():
          pltpu.make_async_copy(
              out_bufs[slot], out_hbm.at[pl.ds(mrow - 2, 1)], sems[slot]
          ).wait()

        pltpu.sync_copy(idx_hbm.at[pl.ds(mrow, 1)], idx_bufs[slot])
        compute_chunk(slot)
        pltpu.async_copy(
            out_bufs[slot], out_hbm.at[pl.ds(mrow, 1)], sems[slot]
        )
      return carry

    lax.fori_loop(0, MROWS_PER_W // 2, body, 0)

    for slot in range(2):
      mrow = wbase + MROWS_PER_W - 2 + slot
      pltpu.make_async_copy(
          out_bufs[slot], out_hbm.at[pl.ds(mrow, 1)], sems[slot]
      ).wait()

  return k(bucket_mtx, table)


def kernel(bucket_mtx, embeddings_table):
  out = _sc_lookup(bucket_mtx, embeddings_table)
  return jnp.transpose(out, (0, 2, 1))


# speedup vs baseline: 110.5997x; 1.4202x over previous
import functools

import jax
import jax.numpy as jnp
from jax import lax
from jax.experimental import pallas as pl
from jax.experimental.pallas import tpu as pltpu
from jax.experimental.pallas import tpu_sc as plsc

L_SIZE = 2048
EMB = 16
NW = 32
CHUNK = L_SIZE
MROWS_PER_W = L_SIZE // NW
GROUPS = CHUNK // 16


def _sc_lookup(bucket_mtx, table):
  mesh = plsc.VectorSubcoreMesh(core_axis_name="c", subcore_axis_name="s")

  @functools.partial(
      pl.kernel,
      mesh=mesh,
      out_type=jax.ShapeDtypeStruct((L_SIZE, EMB, L_SIZE), jnp.float32),
      compiler_params=pltpu.CompilerParams(needs_layout_passes=False),
      scratch_types=[
          pltpu.VMEM((9, EMB), jnp.float32),
          pltpu.VMEM((16 * EMB,), jnp.float32),
          pltpu.VMEM((1, CHUNK), jnp.int32),
          pltpu.VMEM((1, CHUNK), jnp.int32),
          pltpu.VMEM((1, EMB, CHUNK), jnp.float32),
          pltpu.VMEM((1, EMB, CHUNK), jnp.float32),
          pltpu.SemaphoreType.DMA,
          pltpu.SemaphoreType.DMA,
          pltpu.SemaphoreType.DMA,
          pltpu.SemaphoreType.DMA,
      ],
  )
  def k(idx_hbm, tab_hbm, out_hbm, tab2d_v, tab_flat, idx_v0, idx_v1,
        out_v0, out_v1, sem0, sem1, isem0, isem1):
    wid = lax.axis_index("s") * 2 + lax.axis_index("c")
    wbase = wid * MROWS_PER_W
    idx_bufs = (idx_v0, idx_v1)
    out_bufs = (out_v0, out_v1)
    sems = (sem0, sem1)
    isems = (isem0, isem1)

    pltpu.sync_copy(tab_hbm, tab2d_v)
    for e in range(9):
      tab_flat[pl.ds(e * EMB, EMB)] = tab2d_v[e, :]
    lane = lax.iota(jnp.int32, 16)
    tcols = [
        plsc.load_gather(tab_flat, [lane * EMB + k]) for k in range(EMB)
    ]

    def compute_chunk(slot):
      def grp(g2, carry):
        j0 = g2 * 16
        v_idx = idx_bufs[slot][0, pl.ds(j0, 16)]
        for k in range(EMB):
          col = jnp.take_along_axis(tcols[k], v_idx, axis=0)
          out_bufs[slot][0, k, pl.ds(j0, 16)] = col
        return carry

      lax.fori_loop(0, GROUPS, grp, 0, unroll=4)

    for slot in range(2):
      pltpu.async_copy(
          idx_hbm.at[pl.ds(wbase + slot, 1)], idx_bufs[slot], isems[slot]
      )

    def body(t, carry):
      for slot in range(2):
        mrow = wbase + t * 2 + slot

        pltpu.make_async_copy(
            idx_hbm.at[pl.ds(mrow, 1)], idx_bufs[slot], isems[slot]
        ).wait()

        @pl.when(t > 0)
        def _wait():
          pltpu.make_async_copy(
              out_bufs[slot], out_hbm.at[pl.ds(mrow - 2, 1)], sems[slot]
          ).wait()

        compute_chunk(slot)

        @pl.when(mrow + 2 < wbase + MROWS_PER_W)
        def _prefetch():
          pltpu.async_copy(
              idx_hbm.at[pl.ds(mrow + 2, 1)], idx_bufs[slot], isems[slot]
          )

        pltpu.async_copy(
            out_bufs[slot], out_hbm.at[pl.ds(mrow, 1)], sems[slot]
        )
      return carry

    lax.fori_loop(0, MROWS_PER_W // 2, body, 0)

    for slot in range(2):
      mrow = wbase + MROWS_PER_W - 2 + slot
      pltpu.make_async_copy(
          out_bufs[slot], out_hbm.at[pl.ds(mrow, 1)], sems[slot]
      ).wait()

  return k(bucket_mtx, table)


def kernel(bucket_mtx, embeddings_table):
  out = _sc_lookup(bucket_mtx, embeddings_table)
  return jnp.transpose(out, (0, 2, 1))
